# baseline (speedup 1.0000x reference)
import jax
import jax.numpy as jnp
from jax import lax
from jax.experimental import pallas as pl
from jax.experimental.pallas import tpu as pltpu
from jax.experimental.pallas import tpu_sc as plsc

N = 16384
M = 32
L = 16
NV = N // L
CH = 4
NVC = NV // CH
DC_SHIFT = 12
R1_BITS = 12
R1 = 1 << R1_BITS
R23_BITS = 9
R23 = 1 << R23_BITS
IDX_BITS = 14
IDX_MASK = (1 << IDX_BITS) - 1

_LN2 = 0.6931471805599453
_SQRT2 = 1.4142135623730951


def _log_poly(x):
  bits = plsc.bitcast(x, jnp.int32)
  e = jnp.right_shift(bits, 23) - 127
  m = plsc.bitcast(
      jnp.bitwise_or(jnp.bitwise_and(bits, 0x7FFFFF), 0x3F800000),
      jnp.float32)
  big = m > _SQRT2
  m = jnp.where(big, m * 0.5, m)
  e = e + jnp.where(big, 1, 0)
  s = (m - 1.0) / (m + 1.0)
  s2 = s * s
  p = 1.0 + s2 * (1.0 / 3.0 + s2 * (0.2 + s2 * (1.0 / 7.0 + s2 / 9.0)))
  return e.astype(jnp.float32) * _LN2 + 2.0 * s * p


def _sc_body(inp_hbm, out_hbm,
             time_c, risk_c, ev_c, key_a, work_b,
             h1_0, h1_1, h1_2, h1_3,
             o1_0, o1_1, o1_2, o1_3,
             h2, h3,
             o2_0, o2_1, o2_2, o2_3,
             pvec):
  h1 = [h1_0, h1_1, h1_2, h1_3]
  o1 = [o1_0, o1_1, o1_2, o1_3]
  o2 = [o2_0, o2_1, o2_2, o2_3]

  wid = lax.axis_index("s") * 2 + lax.axis_index("c")

  pltpu.sync_copy(inp_hbm.at[0, wid], time_c)
  pltpu.sync_copy(inp_hbm.at[1, wid], risk_c)
  pltpu.sync_copy(inp_hbm.at[2, wid], ev_c)

  zero_i = jnp.zeros((L,), jnp.int32)
  zero_f = jnp.zeros((L,), jnp.float32)
  one_i = jnp.ones((L,), jnp.int32)
  lane_iota = lax.iota(jnp.int32, L)

  def clear_all(refs, nv):
    def body(j, _):
      sl = pl.ds(j * L, L)
      for r in refs:
        r[sl] = zero_i
      return 0
    lax.fori_loop(0, nv, body, 0, unroll=4)

  clear_all(h1, R1 // L)
  clear_all([h2], CH * R23 // L)

  def keygen(i, carry):
    maxv, s1v, sev = carry
    for c in range(CH):
      sl = pl.ds((c * NVC + i) * L, L)
      t = time_c[sl]
      r = risk_c[sl]
      e = ev_c[sl]
      k = 0x3F7FFFFF - plsc.bitcast(t, jnp.int32)
      key_a[sl] = k
      d1 = jnp.bitwise_and(k, R1 - 1)
      plsc.addupdate_scatter(h1[c], [d1], one_i)
      maxv = jnp.maximum(maxv, r)
      s1v = s1v + r * e
      sev = sev + e
    return (maxv, s1v, sev)

  maxv, s1v, sev = lax.fori_loop(
      0, NVC, keygen,
      (jnp.full((L,), -jnp.inf, jnp.float32), zero_f, zero_f), unroll=2)
  gamma = jnp.max(maxv)

  def make_offsets1():
    def body(j, carry):
      sl = pl.ds(j * L, L)
      hv = [h1[c][sl] for c in range(CH)]
      tv = hv[0] + hv[1] + hv[2] + hv[3]
      inc = plsc.cumsum(tv)
      acc = inc - tv + carry
      o1[0][sl] = acc
      for c in range(1, CH):
        acc = acc + hv[c - 1]
        o1[c][sl] = acc
      return carry + jnp.sum(tv)
    lax.fori_loop(0, R1 // L, body, jnp.int32(0), unroll=2)

  make_offsets1()

  def perm1(i, _):
    for c in range(CH):
      sl = pl.ds((c * NVC + i) * L, L)
      k = key_a[sl]
      d = jnp.bitwise_and(k, R1 - 1)
      occ, last = plsc.scan_count(d)
      base = plsc.load_gather(o1[c], [d])
      pos = base + occ - 1
      pack = jnp.bitwise_or(
          jnp.left_shift(jnp.right_shift(k, R1_BITS), IDX_BITS),
          (c * NVC + i) * L + lane_iota)
      plsc.store_scatter(work_b, [pos], pack)
      plsc.store_scatter(o1[c], [d], base + occ, mask=last)
      d2 = jnp.bitwise_and(jnp.right_shift(pack, IDX_BITS), R23 - 1)
      comb = jnp.bitwise_or(
          jnp.left_shift(jnp.right_shift(pos, DC_SHIFT), R23_BITS), d2)
      plsc.addupdate_scatter(h2, [comb], one_i)
    return 0
  lax.fori_loop(0, NVC, perm1, 0)

  def make_offsets23(h_ref, o_refs):
    def body(j, carry):
      osl = pl.ds(j * L, L)
      hs = [h_ref[pl.ds(sc * R23 + j * L, L)] for sc in range(CH)]
      tv = hs[0] + hs[1] + hs[2] + hs[3]
      inc = plsc.cumsum(tv)
      acc = inc - tv + carry
      o_refs[0][osl] = acc
      for sc in range(1, CH):
        acc = acc + hs[sc - 1]
        o_refs[sc][osl] = acc
      return carry + jnp.sum(tv)
    lax.fori_loop(0, R23 // L, body, jnp.int32(0), unroll=2)

  make_offsets23(h2, o2)
  clear_all([h3], CH * R23 // L)

  def perm2(i, _):
    for c in range(CH):
      sl = pl.ds((c * NVC + i) * L, L)
      pack = work_b[sl]
      d = jnp.bitwise_and(jnp.right_shift(pack, IDX_BITS), R23 - 1)
      occ, last = plsc.scan_count(d)
      base = plsc.load_gather(o2[c], [d])
      pos = base + occ - 1
      plsc.store_scatter(key_a, [pos], pack)
      plsc.store_scatter(o2[c], [d], base + occ, mask=last)
      d3 = jnp.bitwise_and(
          jnp.right_shift(pack, IDX_BITS + R23_BITS), R23 - 1)
      comb = jnp.bitwise_or(
          jnp.left_shift(jnp.right_shift(pos, DC_SHIFT), R23_BITS), d3)
      plsc.addupdate_scatter(h3, [comb], one_i)
    return 0
  lax.fori_loop(0, NVC, perm2, 0)

  make_offsets23(h3, o2)

  def perm3(i, _):
    for c in range(CH):
      sl = pl.ds((c * NVC + i) * L, L)
      pack = key_a[sl]
      d = jnp.bitwise_and(
          jnp.right_shift(pack, IDX_BITS + R23_BITS), R23 - 1)
      occ, last = plsc.scan_count(d)
      base = plsc.load_gather(o2[c], [d])
      pos = base + occ - 1
      plsc.store_scatter(work_b, [pos], pack)
      plsc.store_scatter(o2[c], [d], base + occ, mask=last)
    return 0
  lax.fori_loop(0, NVC, perm3, 0)

  def expsum(i, carry):
    vs = list(carry)
    for c in range(CH):
      sl = pl.ds((c * NVC + i) * L, L)
      iv = jnp.bitwise_and(work_b[sl], IDX_MASK)
      r = plsc.load_gather(risk_c, [iv])
      x = jnp.exp(r - gamma)
      time_c[sl] = x
      vs[c] = vs[c] + x
    return tuple(vs)

  vs = lax.fori_loop(0, NVC, expsum, (zero_f,) * CH, unroll=2)
  base_c = [jnp.float32(0.0)]
  for c in range(1, CH):
    base_c.append(base_c[c - 1] + jnp.sum(vs[c - 1]))

  def cox_body(i, carry):
    cs = list(carry[:CH])
    accs = list(carry[CH:])
    for c in range(CH):
      sl = pl.ds((c * NVC + i) * L, L)
      x = time_c[sl]
      iv = jnp.bitwise_and(work_b[sl], IDX_MASK)
      e = plsc.load_gather(ev_c, [iv])
      csum = plsc.cumsum(x) + cs[c]
      lg = _log_poly(csum + 1e-10)
      accs[c] = accs[c] + e * lg
      cs[c] = cs[c] + jnp.sum(x)
    return tuple(cs) + tuple(accs)

  res = lax.fori_loop(
      0, NVC, cox_body,
      tuple(base_c) + (zero_f,) * CH, unroll=2)
  acc2 = res[CH] + res[CH + 1] + res[CH + 2] + res[CH + 3]

  pvec[...] = s1v - acc2 - gamma * sev
  pltpu.sync_copy(pvec, out_hbm.at[wid])


@jax.jit
def _cox_loss(stacked):
  mesh = plsc.VectorSubcoreMesh(core_axis_name="c", subcore_axis_name="s")
  f = pl.kernel(
      _sc_body,
      out_type=jax.ShapeDtypeStruct((M, L), jnp.float32),
      mesh=mesh,
      scratch_types=(
          [
              pltpu.VMEM((N,), jnp.float32),
              pltpu.VMEM((N,), jnp.float32),
              pltpu.VMEM((N,), jnp.float32),
              pltpu.VMEM((N,), jnp.int32),
              pltpu.VMEM((N,), jnp.int32),
          ]
          + [pltpu.VMEM((R1,), jnp.int32) for _ in range(CH)]
          + [pltpu.VMEM((R1,), jnp.int32) for _ in range(CH)]
          + [pltpu.VMEM((CH * R23,), jnp.int32)]
          + [pltpu.VMEM((CH * R23,), jnp.int32)]
          + [pltpu.VMEM((R23,), jnp.int32) for _ in range(CH)]
          + [pltpu.VMEM((L,), jnp.float32)]
      ),
      compiler_params=pltpu.CompilerParams(needs_layout_passes=False),
  )
  out = f(stacked)
  return -(jnp.sum(out) / (N * M))


def kernel(risk_pred, time, event):
  stacked = jnp.stack([time.T, risk_pred.T, event.T])
  return _cox_loss(stacked)


# speedup vs baseline: 1.2298x; 1.2298x over previous
import jax
import jax.numpy as jnp
from jax import lax
from jax.experimental import pallas as pl
from jax.experimental.pallas import tpu as pltpu
from jax.experimental.pallas import tpu_sc as plsc

N = 16384
M = 32
L = 16
NV = N // L
R1_BITS = 12
R1 = 1 << R1_BITS
R23_BITS = 9
R23 = 1 << R23_BITS
IDX_BITS = 14
IDX_MASK = (1 << IDX_BITS) - 1

_LN2 = 0.6931471805599453
_SQRT2 = 1.4142135623730951


def _log_poly(x):
  bits = plsc.bitcast(x, jnp.int32)
  e = jnp.right_shift(bits, 23) - 127
  m = plsc.bitcast(
      jnp.bitwise_or(jnp.bitwise_and(bits, 0x7FFFFF), 0x3F800000),
      jnp.float32)
  big = m > _SQRT2
  m = jnp.where(big, m * 0.5, m)
  e = e + jnp.where(big, 1, 0)
  s = (m - 1.0) / (m + 1.0)
  s2 = s * s
  p = 1.0 + s2 * (1.0 / 3.0 + s2 * (0.2 + s2 * (1.0 / 7.0 + s2 / 9.0)))
  return e.astype(jnp.float32) * _LN2 + 2.0 * s * p


def _sc_body(time_hbm, risk_hbm, ev_hbm, out_hbm,
             time_c, risk_c, ev_c, key_a, work_b,
             hist1, hist2, hist3, off, pvec):
  wid = lax.axis_index("s") * 2 + lax.axis_index("c")

  pltpu.sync_copy(time_hbm.at[wid], time_c)
  pltpu.sync_copy(risk_hbm.at[wid], risk_c)
  pltpu.sync_copy(ev_hbm.at[wid], ev_c)

  zero_i = jnp.zeros((L,), jnp.int32)
  zero_f = jnp.zeros((L,), jnp.float32)
  one_i = jnp.ones((L,), jnp.int32)
  lane_iota = lax.iota(jnp.int32, L)

  def clear(h_ref, nv):
    def body(j, _):
      h_ref[pl.ds(j * L, L)] = zero_i
      return 0
    lax.fori_loop(0, nv, body, 0, unroll=8)

  clear(hist1, R1 // L)
  clear(hist2, R23 // L)
  clear(hist3, R23 // L)

  def keygen(i, carry):
    maxv, s1v, sev = carry
    sl = pl.ds(i * L, L)
    t = time_c[sl]
    r = risk_c[sl]
    e = ev_c[sl]
    k = 0x3F7FFFFF - plsc.bitcast(t, jnp.int32)
    key_a[sl] = k
    d1 = jnp.bitwise_and(k, R1 - 1)
    plsc.addupdate_scatter(hist1, [d1], one_i)
    return (jnp.maximum(maxv, r), s1v + r * e, sev + e)

  maxv, s1v, sev = lax.fori_loop(
      0, NV, keygen, (jnp.full((L,), -jnp.inf, jnp.float32), zero_f, zero_f),
      unroll=4)
  gamma = jnp.max(maxv)

  def hist_scan(h_ref, nv):
    def body(j, carry):
      sl = pl.ds(j * L, L)
      h = h_ref[sl]
      inc = plsc.cumsum(h)
      off[sl] = inc - h + carry
      return carry + jnp.sum(h)
    lax.fori_loop(0, nv, body, jnp.int32(0), unroll=4)

  hist_scan(hist1, R1 // L)

  def perm1(i, _):
    k = key_a[pl.ds(i * L, L)]
    d = jnp.bitwise_and(k, R1 - 1)
    occ, last = plsc.scan_count(d)
    base = plsc.load_gather(off, [d])
    pos = base + occ - 1
    pack = jnp.bitwise_or(
        jnp.left_shift(jnp.right_shift(k, R1_BITS), IDX_BITS),
        i * L + lane_iota)
    plsc.store_scatter(work_b, [pos], pack)
    plsc.store_scatter(off, [d], base + occ, mask=last)
    d2 = jnp.bitwise_and(jnp.right_shift(k, R1_BITS), R23 - 1)
    plsc.addupdate_scatter(hist2, [d2], one_i)
    return 0
  lax.fori_loop(0, NV, perm1, 0, unroll=4)

  hist_scan(hist2, R23 // L)

  def perm2(i, _):
    pack = work_b[pl.ds(i * L, L)]
    d = jnp.bitwise_and(jnp.right_shift(pack, IDX_BITS), R23 - 1)
    occ, last = plsc.scan_count(d)
    base = plsc.load_gather(off, [d])
    pos = base + occ - 1
    plsc.store_scatter(key_a, [pos], pack)
    plsc.store_scatter(off, [d], base + occ, mask=last)
    d3 = jnp.bitwise_and(
        jnp.right_shift(pack, IDX_BITS + R23_BITS), R23 - 1)
    plsc.addupdate_scatter(hist3, [d3], one_i)
    return 0
  lax.fori_loop(0, NV, perm2, 0, unroll=4)

  hist_scan(hist3, R23 // L)

  def perm3(i, _):
    pack = key_a[pl.ds(i * L, L)]
    d = jnp.bitwise_and(jnp.right_shift(pack, IDX_BITS + R23_BITS), R23 - 1)
    occ, last = plsc.scan_count(d)
    base = plsc.load_gather(off, [d])
    pos = base + occ - 1
    plsc.store_scatter(work_b, [pos], pack)
    plsc.store_scatter(off, [d], base + occ, mask=last)
    return 0
  lax.fori_loop(0, NV, perm3, 0, unroll=4)

  def cox_body(i, carry):
    c0, acc2 = carry
    iv = jnp.bitwise_and(work_b[pl.ds(i * L, L)], IDX_MASK)
    r = plsc.load_gather(risk_c, [iv])
    e = plsc.load_gather(ev_c, [iv])
    x = jnp.exp(r - gamma)
    cs_raw = plsc.cumsum(x)
    lg = _log_poly(cs_raw + c0 + 1e-10)
    return (c0 + jnp.sum(x), acc2 + e * lg)

  c0, acc2 = lax.fori_loop(0, NV, cox_body, (jnp.float32(0.0), zero_f),
                           unroll=4)

  pvec[...] = s1v - acc2 - gamma * sev
  pltpu.sync_copy(pvec, out_hbm.at[wid])


@jax.jit
def _cox_loss(time_t, risk_t, ev_t):
  mesh = plsc.VectorSubcoreMesh(core_axis_name="c", subcore_axis_name="s")
  f = pl.kernel(
      _sc_body,
      out_type=jax.ShapeDtypeStruct((M, L), jnp.float32),
      mesh=mesh,
      scratch_types=[
          pltpu.VMEM((N,), jnp.float32),
          pltpu.VMEM((N,), jnp.float32),
          pltpu.VMEM((N,), jnp.float32),
          pltpu.VMEM((N,), jnp.int32),
          pltpu.VMEM((N,), jnp.int32),
          pltpu.VMEM((R1,), jnp.int32),
          pltpu.VMEM((R23,), jnp.int32),
          pltpu.VMEM((R23,), jnp.int32),
          pltpu.VMEM((R1,), jnp.int32),
          pltpu.VMEM((L,), jnp.float32),
      ],
      compiler_params=pltpu.CompilerParams(needs_layout_passes=False),
  )
  out = f(time_t, risk_t, ev_t)
  return -(jnp.sum(out) / (N * M))


def kernel(risk_pred, time, event):
  return _cox_loss(time.T, risk_pred.T, event.T)


# speedup vs baseline: 1.6172x; 1.3150x over previous
import jax
import jax.numpy as jnp
from jax import lax
from jax.experimental import pallas as pl
from jax.experimental.pallas import tpu as pltpu
from jax.experimental.pallas import tpu_sc as plsc

N = 16384
M = 32
L = 16
NV = N // L
NH = NV // 2
R1_BITS = 12
R1 = 1 << R1_BITS
R23_BITS = 9
R23 = 1 << R23_BITS
IDX_BITS = 14
IDX_MASK = (1 << IDX_BITS) - 1

_LN2 = 0.6931471805599453
_SQRT2 = 1.4142135623730951


def _log_poly(x):
  bits = plsc.bitcast(x, jnp.int32)
  e = jnp.right_shift(bits, 23) - 127
  m = plsc.bitcast(
      jnp.bitwise_or(jnp.bitwise_and(bits, 0x7FFFFF), 0x3F800000),
      jnp.float32)
  big = m > _SQRT2
  m = jnp.where(big, m * 0.5, m)
  e = e + jnp.where(big, 1, 0)
  s = (m - 1.0) / (m + 1.0)
  s2 = s * s
  p = 1.0 + s2 * (1.0 / 3.0 + s2 * (0.2 + s2 * (1.0 / 7.0 + s2 / 9.0)))
  return e.astype(jnp.float32) * _LN2 + 2.0 * s * p


def _sc_body(time_hbm, risk_hbm, ev_hbm, out_hbm,
             time_c, risk_c, ev_c, key_a, work_b,
             h1a, h1b, h2a0, h2a1, h2b0, h2b1,
             h3a0, h3a1, h3b0, h3b1, offa, offb, pvec):
  wid = lax.axis_index("s") * 2 + lax.axis_index("c")

  pltpu.sync_copy(time_hbm.at[wid], time_c)
  pltpu.sync_copy(risk_hbm.at[wid], risk_c)
  pltpu.sync_copy(ev_hbm.at[wid], ev_c)

  zero_i = jnp.zeros((L,), jnp.int32)
  zero_f = jnp.zeros((L,), jnp.float32)
  one_i = jnp.ones((L,), jnp.int32)
  lane_iota = lax.iota(jnp.int32, L)
  half_n = jnp.full((L,), N // 2, jnp.int32)

  def clear2(ha_ref, hb_ref, nv):
    def body(j, _):
      ha_ref[pl.ds(j * L, L)] = zero_i
      hb_ref[pl.ds(j * L, L)] = zero_i
      return 0
    lax.fori_loop(0, nv, body, 0, unroll=8)

  clear2(h1a, h1b, R1 // L)
  clear2(h2a0, h2a1, R23 // L)
  clear2(h2b0, h2b1, R23 // L)
  clear2(h3a0, h3a1, R23 // L)
  clear2(h3b0, h3b1, R23 // L)

  def keygen(i, carry):
    maxv, s1v, sev = carry
    sa = pl.ds(i * L, L)
    sb = pl.ds((NH + i) * L, L)
    ta = time_c[sa]
    tb = time_c[sb]
    ra = risk_c[sa]
    rb = risk_c[sb]
    ea = ev_c[sa]
    eb = ev_c[sb]
    ka = 0x3F7FFFFF - plsc.bitcast(ta, jnp.int32)
    kb = 0x3F7FFFFF - plsc.bitcast(tb, jnp.int32)
    key_a[sa] = ka
    key_a[sb] = kb
    plsc.addupdate_scatter(h1a, [jnp.bitwise_and(ka, R1 - 1)], one_i)
    plsc.addupdate_scatter(h1b, [jnp.bitwise_and(kb, R1 - 1)], one_i)
    return (jnp.maximum(jnp.maximum(maxv, ra), rb),
            s1v + ra * ea + rb * eb, sev + ea + eb)

  maxv, s1v, sev = lax.fori_loop(
      0, NH, keygen, (jnp.full((L,), -jnp.inf, jnp.float32), zero_f, zero_f),
      unroll=4)
  gamma = jnp.max(maxv)

  def hist_scan2(ha_ref, hb_ref, nv):
    def body(j, carry):
      sl = pl.ds(j * L, L)
      a = ha_ref[sl]
      b = hb_ref[sl]
      h = a + b
      inc = plsc.cumsum(h)
      base = inc - h + carry
      offa[sl] = base
      offb[sl] = base + a
      return carry + jnp.sum(h)
    lax.fori_loop(0, nv, body, jnp.int32(0), unroll=4)

  def hist_scan4(h0a_ref, h0b_ref, h1a_ref, h1b_ref, nv):
    def body(j, carry):
      sl = pl.ds(j * L, L)
      a = h0a_ref[sl] + h0b_ref[sl]
      b = h1a_ref[sl] + h1b_ref[sl]
      h = a + b
      inc = plsc.cumsum(h)
      base = inc - h + carry
      offa[sl] = base
      offb[sl] = base + a
      return carry + jnp.sum(h)
    lax.fori_loop(0, nv, body, jnp.int32(0), unroll=4)

  hist_scan2(h1a, h1b, R1 // L)

  def perm1(i, _):
    ka = key_a[pl.ds(i * L, L)]
    kb = key_a[pl.ds((NH + i) * L, L)]
    da = jnp.bitwise_and(ka, R1 - 1)
    db = jnp.bitwise_and(kb, R1 - 1)
    occa, lasta = plsc.scan_count(da)
    occb, lastb = plsc.scan_count(db)
    basea = plsc.load_gather(offa, [da])
    baseb = plsc.load_gather(offb, [db])
    posa = basea + occa - 1
    posb = baseb + occb - 1
    packa = jnp.bitwise_or(
        jnp.left_shift(jnp.right_shift(ka, R1_BITS), IDX_BITS),
        i * L + lane_iota)
    packb = jnp.bitwise_or(
        jnp.left_shift(jnp.right_shift(kb, R1_BITS), IDX_BITS),
        (NH + i) * L + lane_iota)
    plsc.store_scatter(work_b, [posa], packa)
    plsc.store_scatter(work_b, [posb], packb)
    plsc.store_scatter(offa, [da], basea + occa, mask=lasta)
    plsc.store_scatter(offb, [db], baseb + occb, mask=lastb)
    d2a = jnp.bitwise_and(jnp.right_shift(ka, R1_BITS), R23 - 1)
    d2b = jnp.bitwise_and(jnp.right_shift(kb, R1_BITS), R23 - 1)
    ma = posa < half_n
    mb = posb < half_n
    plsc.addupdate_scatter(h2a0, [d2a], one_i, mask=ma)
    plsc.addupdate_scatter(h2a1, [d2a], one_i, mask=jnp.logical_not(ma))
    plsc.addupdate_scatter(h2b0, [d2b], one_i, mask=mb)
    plsc.addupdate_scatter(h2b1, [d2b], one_i, mask=jnp.logical_not(mb))
    return 0
  lax.fori_loop(0, NH, perm1, 0, unroll=4)

  hist_scan4(h2a0, h2b0, h2a1, h2b1, R23 // L)

  def perm2(i, _):
    pa = work_b[pl.ds(i * L, L)]
    pb = work_b[pl.ds((NH + i) * L, L)]
    da = jnp.bitwise_and(jnp.right_shift(pa, IDX_BITS), R23 - 1)
    db = jnp.bitwise_and(jnp.right_shift(pb, IDX_BITS), R23 - 1)
    occa, lasta = plsc.scan_count(da)
    occb, lastb = plsc.scan_count(db)
    basea = plsc.load_gather(offa, [da])
    baseb = plsc.load_gather(offb, [db])
    posa = basea + occa - 1
    posb = baseb + occb - 1
    plsc.store_scatter(key_a, [posa], pa)
    plsc.store_scatter(key_a, [posb], pb)
    plsc.store_scatter(offa, [da], basea + occa, mask=lasta)
    plsc.store_scatter(offb, [db], baseb + occb, mask=lastb)
    d3a = jnp.bitwise_and(jnp.right_shift(pa, IDX_BITS + R23_BITS), R23 - 1)
    d3b = jnp.bitwise_and(jnp.right_shift(pb, IDX_BITS + R23_BITS), R23 - 1)
    ma = posa < half_n
    mb = posb < half_n
    plsc.addupdate_scatter(h3a0, [d3a], one_i, mask=ma)
    plsc.addupdate_scatter(h3a1, [d3a], one_i, mask=jnp.logical_not(ma))
    plsc.addupdate_scatter(h3b0, [d3b], one_i, mask=mb)
    plsc.addupdate_scatter(h3b1, [d3b], one_i, mask=jnp.logical_not(mb))
    return 0
  lax.fori_loop(0, NH, perm2, 0, unroll=4)

  hist_scan4(h3a0, h3b0, h3a1, h3b1, R23 // L)

  def perm3(i, _):
    pa = key_a[pl.ds(i * L, L)]
    pb = key_a[pl.ds((NH + i) * L, L)]
    da = jnp.bitwise_and(jnp.right_shift(pa, IDX_BITS + R23_BITS), R23 - 1)
    db = jnp.bitwise_and(jnp.right_shift(pb, IDX_BITS + R23_BITS), R23 - 1)
    occa, lasta = plsc.scan_count(da)
    occb, lastb = plsc.scan_count(db)
    basea = plsc.load_gather(offa, [da])
    baseb = plsc.load_gather(offb, [db])
    posa = basea + occa - 1
    posb = baseb + occb - 1
    plsc.store_scatter(work_b, [posa], pa)
    plsc.store_scatter(work_b, [posb], pb)
    plsc.store_scatter(offa, [da], basea + occa, mask=lasta)
    plsc.store_scatter(offb, [db], baseb + occb, mask=lastb)
    return 0
  lax.fori_loop(0, NH, perm3, 0, unroll=4)

  def cox_body(i, carry):
    c0, acc2 = carry
    iv = jnp.bitwise_and(work_b[pl.ds(i * L, L)], IDX_MASK)
    r = plsc.load_gather(risk_c, [iv])
    e = plsc.load_gather(ev_c, [iv])
    x = jnp.exp(r - gamma)
    cs_raw = plsc.cumsum(x)
    lg = _log_poly(cs_raw + c0 + 1e-10)
    return (c0 + jnp.sum(x), acc2 + e * lg)

  c0, acc2 = lax.fori_loop(0, NV, cox_body, (jnp.float32(0.0), zero_f),
                           unroll=4)

  pvec[...] = s1v - acc2 - gamma * sev
  pltpu.sync_copy(pvec, out_hbm.at[wid])


@jax.jit
def _cox_loss(time_t, risk_t, ev_t):
  mesh = plsc.VectorSubcoreMesh(core_axis_name="c", subcore_axis_name="s")
  f = pl.kernel(
      _sc_body,
      out_type=jax.ShapeDtypeStruct((M, L), jnp.float32),
      mesh=mesh,
      scratch_types=[
          pltpu.VMEM((N,), jnp.float32),
          pltpu.VMEM((N,), jnp.float32),
          pltpu.VMEM((N,), jnp.float32),
          pltpu.VMEM((N,), jnp.int32),
          pltpu.VMEM((N,), jnp.int32),
          pltpu.VMEM((R1,), jnp.int32),
          pltpu.VMEM((R1,), jnp.int32),
          pltpu.VMEM((R23,), jnp.int32),
          pltpu.VMEM((R23,), jnp.int32),
          pltpu.VMEM((R23,), jnp.int32),
          pltpu.VMEM((R23,), jnp.int32),
          pltpu.VMEM((R23,), jnp.int32),
          pltpu.VMEM((R23,), jnp.int32),
          pltpu.VMEM((R23,), jnp.int32),
          pltpu.VMEM((R23,), jnp.int32),
          pltpu.VMEM((R1,), jnp.int32),
          pltpu.VMEM((R1,), jnp.int32),
          pltpu.VMEM((L,), jnp.float32),
      ],
      compiler_params=pltpu.CompilerParams(needs_layout_passes=False),
  )
  out = f(time_t, risk_t, ev_t)
  return -(jnp.sum(out) / (N * M))


def kernel(risk_pred, time, event):
  return _cox_loss(time.T, risk_pred.T, event.T)
